# baseline (speedup 1.0000x reference)
import functools

import jax
import jax.numpy as jnp
from jax import lax
from jax.experimental import pallas as pl
from jax.experimental.pallas import tpu as pltpu
from jax.experimental.pallas import tpu_sc as plsc

BATCH = 16384
FACTORS = 64
NC = 2
NS = 16
NW = NC * NS
BPW = BATCH // NW
CHUNK = 128
NCH = BPW // CHUNK


def _body(u_hbm, i_hbm, j_hbm, eu_hbm, ei_hbm, ub_hbm, ib_hbm, b16_hbm,
          out_i_hbm, out_j_hbm,
          idx_u, idx_i, idx_j, rows_u, rows_i, rows_j,
          ubv, ibiv, ibjv, bv, oiv, ojv, sem):
    cid = lax.axis_index("c")
    sid = lax.axis_index("s")
    wid = sid * NC + cid
    base = wid * BPW

    for k in range(NCH):
        pltpu.sync_copy(u_hbm.at[pl.ds(base + k * CHUNK, CHUNK)], idx_u.at[k])
        pltpu.sync_copy(i_hbm.at[pl.ds(base + k * CHUNK, CHUNK)], idx_i.at[k])
        pltpu.sync_copy(j_hbm.at[pl.ds(base + k * CHUNK, CHUNK)], idx_j.at[k])
    pltpu.sync_copy(b16_hbm, bv)

    copies = []
    for k in range(NCH):
        sl = pl.ds(k * CHUNK, CHUNK)
        copies.append(pltpu.async_copy(eu_hbm.at[idx_u.at[k]], rows_u.at[sl], sem))
        copies.append(pltpu.async_copy(ei_hbm.at[idx_i.at[k]], rows_i.at[sl], sem))
        copies.append(pltpu.async_copy(ei_hbm.at[idx_j.at[k]], rows_j.at[sl], sem))
        copies.append(pltpu.async_copy(ub_hbm.at[idx_u.at[k]], ubv.at[sl], sem))
        copies.append(pltpu.async_copy(ib_hbm.at[idx_i.at[k]], ibiv.at[sl], sem))
        copies.append(pltpu.async_copy(ib_hbm.at[idx_j.at[k]], ibjv.at[sl], sem))
    for c in copies:
        c.wait()

    bvec = bv[...]
    lane = lax.iota(jnp.int32, 16)

    def g_body(g, carry):
        rows = g * 16 + lane
        sl16 = pl.ds(g * 16, 16)
        acc_i = [ubv[sl16] + ibiv[sl16] + bvec, None, None, None]
        acc_j = [ubv[sl16] + ibjv[sl16] + bvec, None, None, None]
        for c in range(FACTORS):
            colv = jnp.full((16,), c, jnp.int32)
            uvec = plsc.load_gather(rows_u, [rows, colv])
            pi = uvec * plsc.load_gather(rows_i, [rows, colv])
            pj = uvec * plsc.load_gather(rows_j, [rows, colv])
            s = c % 4
            acc_i[s] = pi if acc_i[s] is None else acc_i[s] + pi
            acc_j[s] = pj if acc_j[s] is None else acc_j[s] + pj
        oiv[sl16] = (acc_i[0] + acc_i[1]) + (acc_i[2] + acc_i[3])
        ojv[sl16] = (acc_j[0] + acc_j[1]) + (acc_j[2] + acc_j[3])
        return carry

    lax.fori_loop(0, BPW // 16, g_body, 0)

    pltpu.sync_copy(oiv, out_i_hbm.at[pl.ds(base, BPW)])
    pltpu.sync_copy(ojv, out_j_hbm.at[pl.ds(base, BPW)])


@jax.jit
def _pairfm_sc(u, i, j, embed_user, embed_item, ub, ib, b16):
    f32 = jnp.float32
    call = pl.kernel(
        _body,
        out_type=(jax.ShapeDtypeStruct((BATCH,), f32),
                  jax.ShapeDtypeStruct((BATCH,), f32)),
        mesh=plsc.VectorSubcoreMesh(core_axis_name="c", subcore_axis_name="s"),
        compiler_params=pltpu.CompilerParams(needs_layout_passes=False,
                                             use_tc_tiling_on_sc=False),
        scratch_types=[
            pltpu.VMEM((NCH, CHUNK), jnp.int32),
            pltpu.VMEM((NCH, CHUNK), jnp.int32),
            pltpu.VMEM((NCH, CHUNK), jnp.int32),
            pltpu.VMEM((BPW, FACTORS), f32),
            pltpu.VMEM((BPW, FACTORS), f32),
            pltpu.VMEM((BPW, FACTORS), f32),
            pltpu.VMEM((BPW,), f32),
            pltpu.VMEM((BPW,), f32),
            pltpu.VMEM((BPW,), f32),
            pltpu.VMEM((16,), f32),
            pltpu.VMEM((BPW,), f32),
            pltpu.VMEM((BPW,), f32),
            pltpu.SemaphoreType.DMA,
        ],
    )
    return call(u, i, j, embed_user, embed_item, ub, ib, b16)


def kernel(u, i, j, embed_user, embed_item, u_bias, i_bias, bias_):
    u32 = u.astype(jnp.int32)
    i32 = i.astype(jnp.int32)
    j32 = j.astype(jnp.int32)
    ub = u_bias.reshape(-1)
    ib = i_bias.reshape(-1)
    b16 = jnp.broadcast_to(bias_.reshape(()), (16,)).astype(jnp.float32)
    return _pairfm_sc(u32, i32, j32, embed_user, embed_item, ub, ib, b16)
